# baseline (speedup 1.0000x reference)
import functools

import jax
import jax.numpy as jnp
from jax import lax
from jax.experimental import pallas as pl
from jax.experimental.pallas import tpu as pltpu
from jax.experimental.pallas import tpu_sc as plsc

_NUM_CLASSES = 1000000
_HIDDEN = 64
_BATCH = 16384
_DEFAULT = -1

_NC, _NS, _L = 2, 16, 16
_NW = _NC * _NS
_BPW = _BATCH // _NW
_CHUNK = 128
_NCHUNK = _BPW // _CHUNK


def _make_kernel():
    mesh = plsc.VectorSubcoreMesh(core_axis_name="c", subcore_axis_name="s")

    @functools.partial(
        pl.kernel,
        mesh=mesh,
        compiler_params=pltpu.CompilerParams(use_tc_tiling_on_sc=False),
        out_type=jax.ShapeDtypeStruct((_BATCH, _HIDDEN), jnp.float32),
        scratch_types=[
            pltpu.VMEM((_BPW,), jnp.int32),
            pltpu.VMEM((_BPW, _HIDDEN), jnp.float32),
            pltpu.SemaphoreType.DMA,
        ],
    )
    def k(labels_hbm, table_hbm, out_hbm, idx_v, rows_v, sem):
        wid = lax.axis_index("s") * _NC + lax.axis_index("c")
        base = wid * _BPW
        pltpu.sync_copy(labels_hbm.at[pl.ds(base, _BPW)], idx_v)
        for i in range(_BPW // _L):
            v = idx_v[pl.ds(i * _L, _L)]
            idx_v[pl.ds(i * _L, _L)] = jnp.where(v == _DEFAULT, _NUM_CLASSES, v)
        copies = []
        for c in range(_NCHUNK):
            copies.append(
                pltpu.async_copy(
                    table_hbm.at[idx_v.at[pl.ds(c * _CHUNK, _CHUNK)]],
                    rows_v.at[pl.ds(c * _CHUNK, _CHUNK)],
                    sem,
                )
            )
        for cp in copies:
            cp.wait()
        pltpu.sync_copy(rows_v, out_hbm.at[pl.ds(base, _BPW)])

    return k


_gather = _make_kernel()


def kernel(labels, embedding_table):
    return _gather(labels.astype(jnp.int32), embedding_table)


# speedup vs baseline: 1.0479x; 1.0479x over previous
import functools

import jax
import jax.numpy as jnp
from jax import lax
from jax.experimental import pallas as pl
from jax.experimental.pallas import tpu as pltpu
from jax.experimental.pallas import tpu_sc as plsc

_NUM_CLASSES = 1000000
_HIDDEN = 64
_BATCH = 16384
_DEFAULT = -1

_NC, _NS, _L = 2, 16, 16
_NW = _NC * _NS
_BPW = _BATCH // _NW


def _make_kernel():
    mesh = plsc.VectorSubcoreMesh(core_axis_name="c", subcore_axis_name="s")

    @functools.partial(
        pl.kernel,
        mesh=mesh,
        out_type=jax.ShapeDtypeStruct((_BATCH, _HIDDEN), jnp.float32),
        scratch_types=[
            pltpu.VMEM((_BPW,), jnp.int32),
            pltpu.VMEM((_BPW, _HIDDEN), jnp.float32),
            pltpu.SemaphoreType.DMA,
        ],
    )
    def k(labels_hbm, table_hbm, out_hbm, idx_v, rows_v, sem):
        wid = lax.axis_index("s") * _NC + lax.axis_index("c")
        base = wid * _BPW

        pltpu.sync_copy(labels_hbm.at[pl.ds(base, _BPW)], idx_v)

        def body(g, _):
            vec = idx_v[pl.ds(g * _L, _L)]
            vec = jnp.where(vec == _DEFAULT, _NUM_CLASSES, vec)
            for j in range(_L):
                fixed = vec[j]
                pltpu.async_copy(
                    table_hbm.at[pl.ds(fixed, 1)],
                    rows_v.at[pl.ds(g * _L + j, 1)],
                    sem,
                ).wait()
            return ()

        lax.fori_loop(0, _BPW // _L, body, ())
        pltpu.sync_copy(rows_v, out_hbm.at[pl.ds(base, _BPW)])

    return k


_gather = _make_kernel()


def kernel(labels, embedding_table):
    return _gather(labels.astype(jnp.int32), embedding_table)


# speedup vs baseline: 1.7051x; 1.6273x over previous
import functools

import jax
import jax.numpy as jnp
from jax import lax
from jax.experimental import pallas as pl
from jax.experimental.pallas import tpu as pltpu
from jax.experimental.pallas import tpu_sc as plsc

_NUM_CLASSES = 1000000
_HIDDEN = 64
_BATCH = 16384
_DEFAULT = -1

_NC, _NS, _L = 2, 16, 16
_NW = _NC * _NS
_BPW = _BATCH // _NW
_NBATCH = _BPW // _L
_DEPTH = 4


def _make_kernel():
    mesh = plsc.VectorSubcoreMesh(core_axis_name="c", subcore_axis_name="s")

    @functools.partial(
        pl.kernel,
        mesh=mesh,
        out_type=jax.ShapeDtypeStruct((_BATCH, _HIDDEN), jnp.float32),
        scratch_types=[
            pltpu.VMEM((_BPW,), jnp.int32),
            pltpu.VMEM((_BPW, _HIDDEN), jnp.float32),
            pltpu.SemaphoreType.DMA,
        ],
    )
    def k(labels_hbm, table_hbm, out_hbm, idx_v, rows_v, sem):
        wid = lax.axis_index("s") * _NC + lax.axis_index("c")
        base = wid * _BPW

        pltpu.sync_copy(labels_hbm.at[pl.ds(base, _BPW)], idx_v)

        def fire(g):
            vec = idx_v[pl.ds(g * _L, _L)]
            vec = jnp.where(vec == _DEFAULT, _NUM_CLASSES, vec)
            for j in range(_L):
                pltpu.async_copy(
                    table_hbm.at[pl.ds(vec[j], 1)],
                    rows_v.at[pl.ds(g * _L + j, 1)],
                    sem,
                )

        def drain(g):
            for j in range(_L):
                pltpu.make_async_copy(
                    table_hbm.at[pl.ds(0, 1)],
                    rows_v.at[pl.ds(g * _L + j, 1)],
                    sem,
                ).wait()

        for g in range(_DEPTH - 1):
            fire(g)

        def body(g, _):
            fire(g)
            drain(g - (_DEPTH - 1))
            return ()

        lax.fori_loop(_DEPTH - 1, _NBATCH, body, ())

        for g in range(_NBATCH - _DEPTH + 1, _NBATCH):
            drain(g)

        pltpu.sync_copy(rows_v, out_hbm.at[pl.ds(base, _BPW)])

    return k


_gather = _make_kernel()


def kernel(labels, embedding_table):
    return _gather(labels.astype(jnp.int32), embedding_table)
